# baseline (device time: 22581 ns/iter reference)
import jax
import jax.numpy as jnp
from jax import lax
from jax.experimental import pallas as pl
from jax.experimental.pallas import tpu as pltpu

MESH = pl.DeviceIdType.MESH


def kernel(x):
    _, m, n = x.shape
    half = n // 2
    qw = half // 4
    mh = m // 2

    def body(x_ref, out_ref, xv, ysend, yrecv, x2recv, z2recv,
             load_sems, send_sems, recv_sems):
        mx = lax.axis_index("x")
        my = lax.axis_index("y")
        mz = lax.axis_index("z")
        peer_y = (mx, 1 - my, mz)
        nbr_x = (1 - mx, my, mz)
        nbr_z = (mx, my, 1 - mz)

        my_base = my * half
        peer_base = (1 - my) * half
        q_own = 2 * mx + mz
        q_x = 2 * (1 - mx) + mz
        q_z = 2 * mx + (1 - mz)
        q_d = 2 * (1 - mx) + (1 - mz)

        loads = []
        for i, (base, col_q) in enumerate(
            ((peer_base, q_own), (peer_base, q_d),
             (my_base, 0), (my_base, 1), (my_base, 2), (my_base, 3))
        ):
            cp = pltpu.make_async_copy(
                x_ref.at[0, :, pl.ds(base + col_q * qw, qw)],
                xv.at[i],
                load_sems.at[i],
            )
            cp.start()
            loads.append(cp)

        barrier_sem = pltpu.get_barrier_semaphore()
        for nbr in (peer_y, nbr_x, nbr_z):
            pl.semaphore_signal(barrier_sem, inc=1, device_id=nbr,
                                device_id_type=MESH)
        pl.semaphore_wait(barrier_sem, 3)

        def exchange(src, dst, sem_idx, dev):
            r = pltpu.make_async_remote_copy(
                src_ref=src, dst_ref=dst,
                send_sem=send_sems.at[sem_idx],
                recv_sem=recv_sems.at[sem_idx],
                device_id=dev, device_id_type=MESH,
            )
            r.start()
            return r

        def add_quarter_half(q, h, contrib):
            rows = pl.ds(h * mh, mh)
            mine = xv[2 + q, pl.ds(h * mh, mh), :]
            out_ref[rows, pl.ds(q * qw, qw)] = (
                mine + contrib.astype(jnp.float32)
            ).astype(jnp.bfloat16)

        ry = []
        for qi in (0, 1):
            loads[qi].wait()
            for h in (0, 1):
                ysend[qi, h] = xv[qi, pl.ds(h * mh, mh), :].astype(
                    jnp.bfloat16
                )
                ry.append(
                    exchange(ysend.at[qi, h], yrecv.at[qi, h],
                             2 * qi + h, peer_y)
                )

        for cp in loads[2:]:
            cp.wait()

        rx, rz = [], []
        for h in (0, 1):
            ry[h].wait_recv()
            rx.append(exchange(yrecv.at[0, h], x2recv.at[h], 4 + h, nbr_x))
            rz.append(exchange(yrecv.at[0, h], z2recv.at[h], 6 + h, nbr_z))
            add_quarter_half(q_own, h, yrecv[0, h])

        for h in (0, 1):
            ry[2 + h].wait_recv()
            add_quarter_half(q_d, h, yrecv[1, h])

        for h in (0, 1):
            rx[h].wait_recv()
            add_quarter_half(q_x, h, x2recv[h])
        for h in (0, 1):
            rz[h].wait_recv()
            add_quarter_half(q_z, h, z2recv[h])

        for r in ry + rx + rz:
            r.wait_send()

    return pl.pallas_call(
        body,
        out_shape=jax.ShapeDtypeStruct((m, half), jnp.bfloat16),
        in_specs=[pl.BlockSpec(memory_space=pltpu.MemorySpace.HBM)],
        out_specs=pl.BlockSpec(memory_space=pltpu.VMEM),
        scratch_shapes=[
            pltpu.VMEM((6, m, qw), jnp.float32),
            pltpu.VMEM((2, 2, mh, qw), jnp.bfloat16),
            pltpu.VMEM((2, 2, mh, qw), jnp.bfloat16),
            pltpu.VMEM((2, mh, qw), jnp.bfloat16),
            pltpu.VMEM((2, mh, qw), jnp.bfloat16),
            pltpu.SemaphoreType.DMA((6,)),
            pltpu.SemaphoreType.DMA((8,)),
            pltpu.SemaphoreType.DMA((8,)),
        ],
        compiler_params=pltpu.CompilerParams(collective_id=0),
    )(x)


# device time: 22127 ns/iter; 1.0205x vs baseline; 1.0205x over previous
import jax
import jax.numpy as jnp
from jax import lax
from jax.experimental import pallas as pl
from jax.experimental.pallas import tpu as pltpu

MESH = pl.DeviceIdType.MESH


def kernel(x):
    _, m, n = x.shape
    half = n // 2
    qw = half // 4
    mh = m // 2

    xb = x[0].astype(jnp.bfloat16)

    def body(x_ref, out_ref, yrecv, x2recv, z2recv, send_sems, recv_sems):
        mx = lax.axis_index("x")
        my = lax.axis_index("y")
        mz = lax.axis_index("z")
        peer_y = (mx, 1 - my, mz)
        nbr_x = (1 - mx, my, mz)
        nbr_z = (mx, my, 1 - mz)

        my_base = my * half
        peer_base = (1 - my) * half
        q_own = 2 * mx + mz
        q_x = 2 * (1 - mx) + mz
        q_z = 2 * mx + (1 - mz)
        q_d = 2 * (1 - mx) + (1 - mz)

        barrier_sem = pltpu.get_barrier_semaphore()
        for nbr in (peer_y, nbr_x, nbr_z):
            pl.semaphore_signal(barrier_sem, inc=1, device_id=nbr,
                                device_id_type=MESH)
        pl.semaphore_wait(barrier_sem, 3)

        def exchange(src, dst, sem_idx, dev):
            r = pltpu.make_async_remote_copy(
                src_ref=src, dst_ref=dst,
                send_sem=send_sems.at[sem_idx],
                recv_sem=recv_sems.at[sem_idx],
                device_id=dev, device_id_type=MESH,
            )
            r.start()
            return r

        def add_quarter_half(q, h, contrib):
            rows = pl.ds(h * mh, mh)
            mine = x_ref[rows, pl.ds(my_base + q * qw, qw)]
            out_ref[rows, pl.ds(q * qw, qw)] = mine + contrib

        ry = []
        for qi, col_q in ((0, q_own), (1, q_d)):
            for h in (0, 1):
                src = x_ref.at[
                    pl.ds(h * mh, mh), pl.ds(peer_base + col_q * qw, qw)
                ]
                ry.append(
                    exchange(src, yrecv.at[qi, h], 2 * qi + h, peer_y)
                )

        rx, rz = [], []
        for h in (0, 1):
            ry[h].wait_recv()
            rx.append(exchange(yrecv.at[0, h], x2recv.at[h], 4 + h, nbr_x))
            rz.append(exchange(yrecv.at[0, h], z2recv.at[h], 6 + h, nbr_z))
            add_quarter_half(q_own, h, yrecv[0, h])

        for h in (0, 1):
            ry[2 + h].wait_recv()
            add_quarter_half(q_d, h, yrecv[1, h])

        for h in (0, 1):
            rx[h].wait_recv()
            add_quarter_half(q_x, h, x2recv[h])
        for h in (0, 1):
            rz[h].wait_recv()
            add_quarter_half(q_z, h, z2recv[h])

        for r in ry + rx + rz:
            r.wait_send()

    return pl.pallas_call(
        body,
        out_shape=jax.ShapeDtypeStruct((m, half), jnp.bfloat16),
        in_specs=[pl.BlockSpec(memory_space=pltpu.VMEM)],
        out_specs=pl.BlockSpec(memory_space=pltpu.VMEM),
        scratch_shapes=[
            pltpu.VMEM((2, 2, mh, qw), jnp.bfloat16),
            pltpu.VMEM((2, mh, qw), jnp.bfloat16),
            pltpu.VMEM((2, mh, qw), jnp.bfloat16),
            pltpu.SemaphoreType.DMA((8,)),
            pltpu.SemaphoreType.DMA((8,)),
        ],
        compiler_params=pltpu.CompilerParams(collective_id=0),
    )(xb)
